# Initial kernel scaffold; baseline (speedup 1.0000x reference)
import jax
import jax.numpy as jnp
from jax.experimental import pallas as pl


def kernel(charges, cell, positions, neighbor_indices, neighbor_distances):
    raise NotImplementedError("write your pallas kernel here")


# speedup vs baseline: 33.9435x; 33.9435x over previous
import functools

import jax
import jax.numpy as jnp
from jax import lax
from jax.experimental import pallas as pl
from jax.experimental.pallas import tpu as pltpu
from jax.experimental.pallas import tpu_sc as plsc

N_ATOMS = 100000
N_EDGES = 6400000
C = 8
CP = 16
NC = 2
NS = 16
EB = 400
EPT = N_EDGES // (NC * NS)
NB = EPT // EB
N_PAD = 100096
RPT = N_PAD // NS
ZR = 368


def _sc_body(charges, ai, aj, dist, out,
             idxi, idxj, rcp, rows_j, rows_i, zbuf, acc, sem1, sem2):
    cid = lax.axis_index("c")
    sid = lax.axis_index("s")

    def zrow(r, _):
        zbuf[r, :] = jnp.zeros((16,), jnp.float32)
        return 0
    lax.fori_loop(0, ZR, zrow, 0)

    row0 = sid * RPT
    for z in range(RPT // ZR):
        pltpu.sync_copy(zbuf, acc.at[pl.ds(row0 + z * ZR, ZR)])
    plsc.subcore_barrier()

    tile_base = cid * (N_EDGES // NC) + sid * EPT

    def block(b, _):
        base = tile_base + b * EB
        pltpu.sync_copy(ai.at[pl.ds(base, EB)], idxi)
        pltpu.sync_copy(aj.at[pl.ds(base, EB)], idxj)
        pltpu.sync_copy(dist.at[pl.ds(base, EB)], rcp)
        cp1 = pltpu.async_copy(charges.at[idxj], rows_j, sem1)
        cp2 = pltpu.async_copy(charges.at[idxi], rows_i, sem2)
        cp1.wait()
        cp2.wait()

        def scale(k, _):
            dvec = 1.0 / rcp[pl.ds(k * 16, 16)]
            for ei in range(16):
                e = k * 16 + ei
                sv = lax.gather(
                    dvec, jnp.full((16, 1), ei, jnp.int32),
                    lax.GatherDimensionNumbers(offset_dims=(),
                                               collapsed_slice_dims=(0,),
                                               start_index_map=(0,)),
                    (1,), mode=lax.GatherScatterMode.PROMISE_IN_BOUNDS)
                rows_j[e, :] = rows_j[e, :] * sv
                rows_i[e, :] = rows_i[e, :] * sv
            return 0
        lax.fori_loop(0, EB // 16, scale, 0)

        pltpu.sync_copy(rows_j, acc.at[idxi], add=True)
        pltpu.sync_copy(rows_i, acc.at[idxj], add=True)
        return 0

    lax.fori_loop(0, NB, block, 0)

    plsc.subcore_barrier()
    out_base = cid * N_PAD + row0
    pltpu.sync_copy(acc.at[pl.ds(row0, RPT)], out.at[pl.ds(out_base, RPT)])


def _combine(a_ref, b_ref, o_ref):
    o_ref[...] = (a_ref[...] + b_ref[...]) * 0.5


@jax.jit
def _impl(charges, neighbor_indices, neighbor_distances):
    ai = neighbor_indices[:, 0].astype(jnp.int32)
    aj = neighbor_indices[:, 1].astype(jnp.int32)
    charges_p = jnp.concatenate([charges, jnp.zeros_like(charges)], axis=1)

    mesh = plsc.VectorSubcoreMesh(core_axis_name="c", subcore_axis_name="s")
    sck = pl.kernel(
        _sc_body,
        out_type=jax.ShapeDtypeStruct((NC * N_PAD, CP), jnp.float32),
        mesh=mesh,
        scratch_types=[
            pltpu.VMEM((EB,), jnp.int32),
            pltpu.VMEM((EB,), jnp.int32),
            pltpu.VMEM((EB,), jnp.float32),
            pltpu.VMEM((EB, CP), jnp.float32),
            pltpu.VMEM((EB, CP), jnp.float32),
            pltpu.VMEM((ZR, CP), jnp.float32),
            pltpu.VMEM_SHARED((N_PAD, CP), jnp.float32),
            pltpu.SemaphoreType.DMA,
            pltpu.SemaphoreType.DMA,
        ],
        compiler_params=pltpu.CompilerParams(use_tc_tiling_on_sc=False),
    )
    part = sck(charges_p, ai, aj, neighbor_distances)

    a = part[:N_PAD].reshape(12512, 128)
    b = part[N_PAD:].reshape(12512, 128)
    pot = pl.pallas_call(
        _combine,
        out_shape=jax.ShapeDtypeStruct((12512, 128), jnp.float32),
    )(a, b)
    return pot.reshape(N_PAD, CP)[:N_ATOMS, :C]


def kernel(charges, cell, positions, neighbor_indices, neighbor_distances):
    return _impl(charges, neighbor_indices, neighbor_distances)
